# baseline (speedup 1.0000x reference)
import functools

import jax
import jax.numpy as jnp
import numpy as np
from jax import lax
from jax.experimental import pallas as pl
from jax.experimental.pallas import tpu as pltpu
from jax.experimental.pallas import tpu_sc as plsc

VOCAB = 1000000
ACTIVE = 1000000
D = 64
NB = 4
B = 1024
S = 200
DQK = D // 4
BS = B * S
NIDS = BS * NB

NC, NS = 2, 16
NW = NC * NS
IDS_PER_W = NIDS // NW
CHUNK = 512
NCHUNK = IDS_PER_W // CHUNK


def _sc_gather_body(tab_hbm, idx_hbm, out_hbm, idx_v, rows_v, gsem, wsem):
    wid = lax.axis_index("s") * NC + lax.axis_index("c")
    base = wid * IDS_PER_W

    def step(c, carry):
        off = base + c * CHUNK
        pltpu.sync_copy(idx_hbm.at[pl.ds(off, CHUNK)], idx_v)
        pltpu.async_copy(tab_hbm.at[idx_v], rows_v, gsem).wait()
        pltpu.async_copy(rows_v, out_hbm.at[pl.ds(off, CHUNK)], wsem).wait()
        return carry

    lax.fori_loop(0, NCHUNK, step, 0)


@functools.partial(jax.jit, static_argnums=())
def _sc_gather(table, flat_ids):
    k = pl.kernel(
        _sc_gather_body,
        out_type=jax.ShapeDtypeStruct((NIDS, D), jnp.float32),
        mesh=plsc.VectorSubcoreMesh(core_axis_name="c", subcore_axis_name="s"),
        scratch_types=[
            pltpu.VMEM((CHUNK,), jnp.int32),
            pltpu.VMEM((CHUNK, D), jnp.float32),
            pltpu.SemaphoreType.DMA,
            pltpu.SemaphoreType.DMA,
        ],
        compiler_params=pltpu.CompilerParams(use_tc_tiling_on_sc=False),
    )
    return k(table, flat_ids)


TB = 512
NBLK = BS // TB


def _tc_attn_body(g_ref, mod_ref, wq_ref, bq_ref, wk4_ref, bk4_ref,
                  sel_ref, wsel_ref, fold_ref, wo_ref, bo_ref, out_ref):
    f32 = jnp.float32
    hp = lax.Precision.HIGHEST
    g = g_ref[...]
    gm = g * mod_ref[...]
    e0 = g[:, :D]
    q = lax.dot_general(e0, wq_ref[...], (((1,), (0,)), ((), ())),
                        precision=hp, preferred_element_type=f32) + bq_ref[...]
    kall = lax.dot_general(gm, wk4_ref[...], (((1,), (0,)), ((), ())),
                           precision=hp, preferred_element_type=f32) + bk4_ref[...]
    q4 = jnp.concatenate([q, q, q, q], axis=1)
    scores = lax.dot_general(kall * q4, sel_ref[...], (((1,), (0,)), ((), ())),
                             precision=hp, preferred_element_type=f32)
    m = jnp.max(scores, axis=1, keepdims=True)
    ex = jnp.exp(scores - m)
    w = ex / jnp.sum(ex, axis=1, keepdims=True)
    w256 = lax.dot_general(w, wsel_ref[...], (((1,), (0,)), ((), ())),
                           precision=hp, preferred_element_type=f32)
    coll = lax.dot_general(gm * w256, fold_ref[...], (((1,), (0,)), ((), ())),
                           precision=hp, preferred_element_type=f32)
    out_ref[...] = lax.dot_general(coll, wo_ref[...], (((1,), (0,)), ((), ())),
                                   precision=hp, preferred_element_type=f32) + bo_ref[...]


def _tc_attn(g2, mod256, Wq, bq2, Wk4, bk4, sel, wsel, fold, Wo, bo2):
    full = lambda shape: pl.BlockSpec(shape, lambda i: (0,) * len(shape))
    return pl.pallas_call(
        _tc_attn_body,
        grid=(NBLK,),
        in_specs=[
            pl.BlockSpec((TB, NB * D), lambda i: (i, 0)),
            full((1, NB * D)),
            full((D, DQK)),
            full((1, DQK)),
            full((NB * D, NB * DQK)),
            full((1, NB * DQK)),
            full((NB * DQK, NB)),
            full((NB, NB * D)),
            full((NB * D, D)),
            full((D, D)),
            full((1, D)),
        ],
        out_specs=pl.BlockSpec((TB, D), lambda i: (i, 0)),
        out_shape=jax.ShapeDtypeStruct((BS, D), jnp.float32),
    )(g2, mod256, Wq, bq2, Wk4, bk4, sel, wsel, fold, Wo, bo2)


def kernel(inputs, table, branch_basis, Wq, bq, Wk, bk, Wo, bo):
    flat_ids = jnp.minimum(inputs.reshape(-1), ACTIVE - 1)
    g = _sc_gather(table, flat_ids)
    g2 = g.reshape(BS, NB * D)

    scale = jax.nn.sigmoid(jnp.mean(branch_basis[:NB, :], axis=-1))
    mods = 0.9 + 0.2 * scale
    mod256 = jnp.repeat(mods, D).reshape(1, NB * D)
    eye_nb = jnp.eye(NB, dtype=jnp.float32)
    inv_sqrt = 1.0 / np.sqrt(np.float32(DQK))
    Wk4 = jnp.kron(eye_nb, Wk) * inv_sqrt
    bk4 = jnp.tile(bk, NB).reshape(1, NB * DQK) * inv_sqrt
    sel = jnp.kron(eye_nb, jnp.ones((DQK, 1), jnp.float32))
    wsel = jnp.kron(eye_nb, jnp.ones((1, D), jnp.float32))
    fold = jnp.tile(jnp.eye(D, dtype=jnp.float32), (NB, 1))

    out = _tc_attn(g2, mod256, Wq, bq.reshape(1, DQK), Wk4, bk4,
                   sel, wsel, fold, Wo, bo.reshape(1, D))
    return out.reshape(B, S, D)


# speedup vs baseline: 1.4338x; 1.4338x over previous
import jax
import jax.numpy as jnp
from jax import lax
from jax.experimental import pallas as pl
from jax.experimental.pallas import tpu as pltpu
from jax.experimental.pallas import tpu_sc as plsc

VOCAB = 1000000
ACTIVE = 1000000
D = 64
NB = 4
B = 1024
S = 200
DQK = D // 4
BS = B * S
NIDS = BS * NB

NC, NS = 2, 16
NW = NC * NS
IDS_PER_W = NIDS // NW
CHUNK = 512
NCHUNK = IDS_PER_W // CHUNK


def _sc_gather_body(tab_hbm, idx_hbm, out_hbm,
                    idx_v, rows0, rows1, gsem0, gsem1, wsem0, wsem1):
    wid = lax.axis_index("s") * NC + lax.axis_index("c")
    base = wid * IDS_PER_W
    rows = (rows0, rows1)
    gsem = (gsem0, gsem1)
    wsem = (wsem0, wsem1)

    pltpu.sync_copy(idx_hbm.at[pl.ds(base, IDS_PER_W)], idx_v)

    def idx_at(c):
        return idx_v.at[pl.ds(c * CHUNK, CHUNK)]

    for b in range(2):
        pltpu.async_copy(tab_hbm.at[idx_at(b)], rows[b], gsem[b])

    def step(g, carry):
        for b in range(2):
            c = 2 * g + b
            off = base + c * CHUNK
            pltpu.make_async_copy(tab_hbm.at[idx_at(c)], rows[b], gsem[b]).wait()
            pltpu.async_copy(rows[b], out_hbm.at[pl.ds(off, CHUNK)], wsem[b])

            @pl.when(c + 2 < NCHUNK)
            def _():
                pltpu.make_async_copy(
                    rows[b], out_hbm.at[pl.ds(off, CHUNK)], wsem[b]).wait()
                pltpu.async_copy(tab_hbm.at[idx_at(c + 2)], rows[b], gsem[b])
        return carry

    lax.fori_loop(0, NCHUNK // 2, step, 0)

    for b in range(2):
        c = NCHUNK - 2 + b
        off = base + c * CHUNK
        pltpu.make_async_copy(rows[b], out_hbm.at[pl.ds(off, CHUNK)], wsem[b]).wait()


def _sc_gather(table, flat_ids):
    k = pl.kernel(
        _sc_gather_body,
        out_type=jax.ShapeDtypeStruct((NIDS, D), jnp.float32),
        mesh=plsc.VectorSubcoreMesh(core_axis_name="c", subcore_axis_name="s"),
        scratch_types=[
            pltpu.VMEM((IDS_PER_W,), jnp.int32),
            pltpu.VMEM((CHUNK, D), jnp.float32),
            pltpu.VMEM((CHUNK, D), jnp.float32),
            pltpu.SemaphoreType.DMA,
            pltpu.SemaphoreType.DMA,
            pltpu.SemaphoreType.DMA,
            pltpu.SemaphoreType.DMA,
        ],
        compiler_params=pltpu.CompilerParams(use_tc_tiling_on_sc=False),
    )
    return k(table, flat_ids)


TBT = 3200
NBLK = BS // TBT
RPB = 16


def _tc_attn_body(g_ref, mods_ref, wq_ref, bq_ref, wk_ref, wo_ref, bo_ref,
                  out_ref):
    f32 = jnp.float32
    hp = lax.Precision.HIGHEST
    mm = lambda a, b: lax.dot_general(a, b, (((1,), (0,)), ((), ())),
                                      precision=hp, preferred_element_type=f32)
    x = g_ref[...]
    x3 = x.reshape(TBT, 2, 128)
    e01 = x3[:, 0, :]
    e23 = x3[:, 1, :]
    es = (e01[:, :D], e01[:, D:], e23[:, :D], e23[:, D:])
    q = mm(es[0], wq_ref[...]) + bq_ref[...]
    sc = [jnp.sum(mm(es[n], wk_ref[...]) * q, axis=1, keepdims=True)
          * (mods_ref[n] * 0.25) for n in range(NB)]
    m = jnp.maximum(jnp.maximum(sc[0], sc[1]), jnp.maximum(sc[2], sc[3]))
    u = [jnp.exp(s - m) for s in sc]
    den = u[0] + u[1] + u[2] + u[3]
    coll = sum((u[n] / den * mods_ref[n]) * es[n] for n in range(NB))
    out = mm(coll, wo_ref[...]) + bo_ref[...]
    out_ref[...] = out.reshape(RPB, S, D)


def _tc_attn(g2, mods, Wq, bq2, Wk, Wo, bo2):
    full = lambda shape: pl.BlockSpec(shape, lambda i: (0,) * len(shape))
    return pl.pallas_call(
        _tc_attn_body,
        grid=(NBLK,),
        in_specs=[
            pl.BlockSpec((2 * TBT, 128), lambda i: (i, 0)),
            pl.BlockSpec(memory_space=pltpu.SMEM),
            full((D, DQK)),
            full((1, DQK)),
            full((D, DQK)),
            full((D, D)),
            full((1, D)),
        ],
        out_specs=pl.BlockSpec((RPB, S, D), lambda i: (i, 0, 0)),
        out_shape=jax.ShapeDtypeStruct((B, S, D), jnp.float32),
    )(g2, mods, Wq, bq2, Wk, Wo, bo2)


def kernel(inputs, table, branch_basis, Wq, bq, Wk, bk, Wo, bo):
    del bk
    flat_ids = jnp.minimum(inputs.reshape(-1), ACTIVE - 1)
    g = _sc_gather(table, flat_ids)
    g2 = g.reshape(NIDS // 2, 2 * D)

    scale = jax.nn.sigmoid(jnp.mean(branch_basis[:NB, :], axis=-1))
    mods = 0.9 + 0.2 * scale

    return _tc_attn(g2, mods, Wq, bq.reshape(1, DQK), Wk, Wo,
                    bo.reshape(1, D))


# speedup vs baseline: 1.8384x; 1.2822x over previous
import jax
import jax.numpy as jnp
from jax import lax
from jax.experimental import pallas as pl
from jax.experimental.pallas import tpu as pltpu
from jax.experimental.pallas import tpu_sc as plsc

VOCAB = 1000000
ACTIVE = 1000000
D = 64
NB = 4
B = 1024
S = 200
DQK = D // 4
BS = B * S
NIDS = BS * NB

NC, NS = 2, 16
NW = NC * NS
IDS_PER_W = NIDS // NW
CHUNK = 512
NCHUNK = IDS_PER_W // CHUNK


def _sc_gather_body(tab_hbm, idx_hbm, out_hbm,
                    idx_v, rows0, rows1, gsem0, gsem1, wsem0, wsem1):
    wid = lax.axis_index("s") * NC + lax.axis_index("c")
    base = wid * IDS_PER_W
    rows = (rows0, rows1)
    gsem = (gsem0, gsem1)
    wsem = (wsem0, wsem1)

    pltpu.sync_copy(idx_hbm.at[pl.ds(base, IDS_PER_W)], idx_v)

    def idx_at(c):
        return idx_v.at[pl.ds(c * CHUNK, CHUNK)]

    for b in range(2):
        pltpu.async_copy(tab_hbm.at[idx_at(b)], rows[b], gsem[b])

    def step(g, carry):
        for b in range(2):
            c = 2 * g + b
            off = base + c * CHUNK
            pltpu.make_async_copy(tab_hbm.at[idx_at(c)], rows[b], gsem[b]).wait()
            pltpu.async_copy(rows[b], out_hbm.at[pl.ds(off, CHUNK)], wsem[b])

            @pl.when(c + 2 < NCHUNK)
            def _():
                pltpu.make_async_copy(
                    rows[b], out_hbm.at[pl.ds(off, CHUNK)], wsem[b]).wait()
                pltpu.async_copy(tab_hbm.at[idx_at(c + 2)], rows[b], gsem[b])
        return carry

    lax.fori_loop(0, NCHUNK // 2, step, 0)

    for b in range(2):
        c = NCHUNK - 2 + b
        off = base + c * CHUNK
        pltpu.make_async_copy(rows[b], out_hbm.at[pl.ds(off, CHUNK)], wsem[b]).wait()


def _sc_gather(table, flat_ids):
    k = pl.kernel(
        _sc_gather_body,
        out_type=jax.ShapeDtypeStruct((NIDS, D), jnp.float32),
        mesh=plsc.VectorSubcoreMesh(core_axis_name="c", subcore_axis_name="s"),
        scratch_types=[
            pltpu.VMEM((IDS_PER_W,), jnp.int32),
            pltpu.VMEM((CHUNK, D), jnp.float32),
            pltpu.VMEM((CHUNK, D), jnp.float32),
            pltpu.SemaphoreType.DMA,
            pltpu.SemaphoreType.DMA,
            pltpu.SemaphoreType.DMA,
            pltpu.SemaphoreType.DMA,
        ],
        compiler_params=pltpu.CompilerParams(use_tc_tiling_on_sc=False),
    )
    return k(table, flat_ids)


TBT = 3200
NBLK = BS // TBT
RPB = 16


def _tc_attn_body(e0_ref, e1_ref, e2_ref, e3_ref, mods_ref, wq_ref, bq_ref,
                  wkt_ref, wo_ref, bo_ref, out_ref):
    f32 = jnp.float32
    mm = lambda a, b: lax.dot_general(a, b, (((1,), (0,)), ((), ())),
                                      preferred_element_type=f32)
    es = (e0_ref[...], e1_ref[...], e2_ref[...], e3_ref[...])
    q = mm(es[0], wq_ref[...]) + bq_ref[...]
    r = mm(q, wkt_ref[...])
    ones = jnp.ones((D, 1), f32)
    sc = [mm(es[n] * r, ones) * (mods_ref[n] * 0.25) for n in range(NB)]
    m = jnp.maximum(jnp.maximum(sc[0], sc[1]), jnp.maximum(sc[2], sc[3]))
    u = [jnp.exp(s - m) for s in sc]
    rden = 1.0 / (u[0] + u[1] + u[2] + u[3])
    coll = sum((u[n] * rden * mods_ref[n]) * es[n] for n in range(NB))
    out = mm(coll, wo_ref[...]) + bo_ref[...]
    out_ref[...] = out.reshape(RPB, S, D)


def _tc_attn(g, mods, Wq, bq2, WkT, Wo, bo2):
    full = lambda shape: pl.BlockSpec(shape, lambda i: (0,) * len(shape))
    plane = BS // TBT
    espec = lambda n: pl.BlockSpec((TBT, D), lambda i, n=n: (n * plane + i, 0))
    return pl.pallas_call(
        _tc_attn_body,
        grid=(NBLK,),
        in_specs=[
            espec(0), espec(1), espec(2), espec(3),
            pl.BlockSpec(memory_space=pltpu.SMEM),
            full((D, DQK)),
            full((1, DQK)),
            full((DQK, D)),
            full((D, D)),
            full((1, D)),
        ],
        out_specs=pl.BlockSpec((RPB, S, D), lambda i: (i, 0, 0)),
        out_shape=jax.ShapeDtypeStruct((B, S, D), jnp.float32),
    )(g, g, g, g, mods, Wq, bq2, WkT, Wo, bo2)


def kernel(inputs, table, branch_basis, Wq, bq, Wk, bk, Wo, bo):
    del bk
    ids = jnp.minimum(inputs.reshape(BS, NB), ACTIVE - 1)
    flat_ids = ids.T.reshape(-1)
    g = _sc_gather(table, flat_ids)

    scale = jax.nn.sigmoid(jnp.mean(branch_basis[:NB, :], axis=-1))
    mods = 0.9 + 0.2 * scale

    return _tc_attn(g, mods, Wq, bq.reshape(1, DQK), Wk.T, Wo,
                    bo.reshape(1, D))


# speedup vs baseline: 1.9350x; 1.0525x over previous
import jax
import jax.numpy as jnp
from jax import lax
from jax.experimental import pallas as pl
from jax.experimental.pallas import tpu as pltpu
from jax.experimental.pallas import tpu_sc as plsc

VOCAB = 1000000
ACTIVE = 1000000
D = 64
D2 = 2 * D
NB = 4
B = 1024
S = 200
DQK = D // 4
BS = B * S
NIDS = BS * NB

NC, NS = 2, 16
NW = NC * NS
IDS_PER_W = NIDS // NW
CHUNK = 256
NCHUNK = IDS_PER_W // CHUNK


def _sc_gather_body(tab_hbm, idx_hbm, out_hbm,
                    idx_v, rows0, rows1, gsem0, gsem1, wsem0, wsem1):
    wid = lax.axis_index("s") * NC + lax.axis_index("c")
    base = wid * IDS_PER_W
    rows = (rows0, rows1)
    gsem = (gsem0, gsem1)
    wsem = (wsem0, wsem1)

    pltpu.sync_copy(idx_hbm.at[pl.ds(base, IDS_PER_W)], idx_v)

    def idx_at(c):
        return idx_v.at[pl.ds(c * CHUNK, CHUNK)]

    for b in range(2):
        pltpu.async_copy(tab_hbm.at[idx_at(b)], rows[b], gsem[b])

    def step(g, carry):
        for b in range(2):
            c = 2 * g + b
            off = base + c * CHUNK
            pltpu.make_async_copy(tab_hbm.at[idx_at(c)], rows[b], gsem[b]).wait()
            pltpu.async_copy(rows[b], out_hbm.at[pl.ds(off, CHUNK)], wsem[b])

            @pl.when(c + 2 < NCHUNK)
            def _():
                pltpu.make_async_copy(
                    rows[b], out_hbm.at[pl.ds(off, CHUNK)], wsem[b]).wait()
                pltpu.async_copy(tab_hbm.at[idx_at(c + 2)], rows[b], gsem[b])
        return carry

    lax.fori_loop(0, NCHUNK // 2, step, 0)

    for b in range(2):
        c = NCHUNK - 2 + b
        off = base + c * CHUNK
        pltpu.make_async_copy(rows[b], out_hbm.at[pl.ds(off, CHUNK)], wsem[b]).wait()


def _sc_gather(table2, flat_ids):
    k = pl.kernel(
        _sc_gather_body,
        out_type=jax.ShapeDtypeStruct((NIDS, D2), jnp.float32),
        mesh=plsc.VectorSubcoreMesh(core_axis_name="c", subcore_axis_name="s"),
        scratch_types=[
            pltpu.VMEM((IDS_PER_W,), jnp.int32),
            pltpu.VMEM((CHUNK, D2), jnp.float32),
            pltpu.VMEM((CHUNK, D2), jnp.float32),
            pltpu.SemaphoreType.DMA,
            pltpu.SemaphoreType.DMA,
            pltpu.SemaphoreType.DMA,
            pltpu.SemaphoreType.DMA,
        ],
    )
    return k(table2, flat_ids)


TBT = 3200
NBLK = BS // TBT
RPB = 16


def _tc_attn_body(e0_ref, e1_ref, e2_ref, e3_ref, mods_ref, wq_ref, bq_ref,
                  wkt_ref, wo_ref, bo_ref, out_ref):
    f32 = jnp.float32
    mm = lambda a, b: lax.dot_general(a, b, (((1,), (0,)), ((), ())),
                                      preferred_element_type=f32)
    es = (e0_ref[...], e1_ref[...], e2_ref[...], e3_ref[...])
    q = mm(es[0], wq_ref[...]) + bq_ref[...]
    r = mm(q, wkt_ref[...])
    ones = jnp.ones((D2, 1), f32)
    sc = [mm(es[n] * r, ones) * (mods_ref[n] * 0.25) for n in range(NB)]
    m = jnp.maximum(jnp.maximum(sc[0], sc[1]), jnp.maximum(sc[2], sc[3]))
    u = [jnp.exp(s - m) for s in sc]
    rden = 1.0 / (u[0] + u[1] + u[2] + u[3])
    coll = sum((u[n] * rden * mods_ref[n]) * es[n] for n in range(NB))
    out = mm(coll, wo_ref[...]) + bo_ref[...]
    out_ref[...] = out.reshape(RPB, S, D)


def _tc_attn(g, mods, Wq2, bq2, WkT2, Wo2, bo2):
    full = lambda shape: pl.BlockSpec(shape, lambda i: (0,) * len(shape))
    plane = BS // TBT
    espec = lambda n: pl.BlockSpec((TBT, D2), lambda i, n=n: (n * plane + i, 0))
    return pl.pallas_call(
        _tc_attn_body,
        grid=(NBLK,),
        in_specs=[
            espec(0), espec(1), espec(2), espec(3),
            pl.BlockSpec(memory_space=pltpu.SMEM),
            full((D2, DQK)),
            full((1, DQK)),
            full((DQK, D2)),
            full((D2, D)),
            full((1, D)),
        ],
        out_specs=pl.BlockSpec((RPB, S, D), lambda i: (i, 0, 0)),
        out_shape=jax.ShapeDtypeStruct((B, S, D), jnp.float32),
    )(g, g, g, g, mods, Wq2, bq2, WkT2, Wo2, bo2)


def kernel(inputs, table, branch_basis, Wq, bq, Wk, bk, Wo, bo):
    del bk
    table2 = jnp.concatenate([table, table], axis=1)
    ids = jnp.minimum(inputs.reshape(BS, NB), ACTIVE - 1)
    flat_ids = ids.T.reshape(-1)
    g = _sc_gather(table2, flat_ids)

    scale = jax.nn.sigmoid(jnp.mean(branch_basis[:NB, :], axis=-1))
    mods = 0.9 + 0.2 * scale

    Wq2 = jnp.concatenate([Wq, Wq], axis=0) * 0.5
    WkT2 = jnp.concatenate([Wk.T, Wk.T], axis=1) * 0.5
    Wo2 = jnp.concatenate([Wo, Wo], axis=0) * 0.5

    return _tc_attn(g, mods, Wq2, bq.reshape(1, DQK), WkT2, Wo2,
                    bo.reshape(1, D))
